# baseline (device time: 19003 ns/iter reference)
import jax
import jax.numpy as jnp
from jax import lax
from jax.experimental import pallas as pl
from jax.experimental.pallas import tpu as pltpu

N_DEV = 16
LOG_N = 4
E_LOCAL = 2
N_EXPERTS = 32
N_TOK = 256
D_OUT = 256
H = 4


def kernel(x, router_W, route_idx, expert_W):
    def body(x_ref, rw_ref, idx_ref, ew_ref, out_ref,
             send_ref, recv_ref, send_sems, recv_sems):
        my = lax.axis_index("i")
        partners = [my ^ (1 << k) for k in range(LOG_N)]

        barrier_sem = pltpu.get_barrier_semaphore()
        for p in partners:
            pl.semaphore_signal(barrier_sem, inc=1, device_id=(p,),
                                device_id_type=pl.DeviceIdType.MESH)

        xv = x_ref[:, :]
        scores = jnp.dot(xv, rw_ref[:, :], preferred_element_type=jnp.float32)
        s_max = jnp.max(scores, axis=-1, keepdims=True)
        probs = jnp.exp(scores - s_max)
        probs = probs / jnp.sum(probs, axis=-1, keepdims=True)

        idx0 = idx_ref[:, 0:1]
        idx1 = idx_ref[:, 1:2]
        eids = lax.broadcasted_iota(jnp.int32, (N_TOK, N_EXPERTS), 1)
        g0 = jnp.sum(probs * (eids == idx0).astype(jnp.float32),
                     axis=1, keepdims=True)
        g1 = jnp.sum(probs * (eids == idx1).astype(jnp.float32),
                     axis=1, keepdims=True)
        gs = g0 + g1

        acc = jnp.zeros((N_TOK, D_OUT), jnp.float32)
        for j in range(E_LOCAL):
            e_glob = my * E_LOCAL + j
            p_e = jnp.sum(probs * (eids == e_glob).astype(jnp.float32),
                          axis=1, keepdims=True)
            sel = jnp.logical_or(idx0 == e_glob, idx1 == e_glob)
            gate = jnp.where(sel, p_e / gs, 0.0)
            y = jnp.dot(xv, ew_ref[j, :, :],
                        preferred_element_type=jnp.float32)
            acc = acc + gate * y
        out_ref[:, :] = acc

        pl.semaphore_wait(barrier_sem, LOG_N)

        rows = N_TOK // H

        def make(k, h):
            return pltpu.make_async_remote_copy(
                src_ref=send_ref.at[k, h],
                dst_ref=recv_ref.at[k, h],
                send_sem=send_sems.at[k, h],
                recv_sem=recv_sems.at[k, h],
                device_id=(partners[k],),
                device_id_type=pl.DeviceIdType.MESH,
            )

        descs = {}
        for h in range(H):
            send_ref[0, h, :, :] = out_ref[pl.ds(h * rows, rows), :].astype(
                jnp.bfloat16)
            descs[(0, h)] = make(0, h)
            descs[(0, h)].start()
        for k in range(LOG_N):
            for h in range(H):
                descs[(k, h)].wait()
                out_ref[pl.ds(h * rows, rows), :] = (
                    out_ref[pl.ds(h * rows, rows), :]
                    + recv_ref[k, h, :, :].astype(jnp.float32)
                )
                if k + 1 < LOG_N:
                    send_ref[k + 1, h, :, :] = out_ref[
                        pl.ds(h * rows, rows), :].astype(jnp.bfloat16)
                    descs[(k + 1, h)] = make(k + 1, h)
                    descs[(k + 1, h)].start()

    return pl.pallas_call(
        body,
        out_shape=jax.ShapeDtypeStruct((N_TOK, D_OUT), jnp.float32),
        in_specs=[
            pl.BlockSpec(memory_space=pltpu.VMEM),
            pl.BlockSpec(memory_space=pltpu.VMEM),
            pl.BlockSpec(memory_space=pltpu.VMEM),
            pl.BlockSpec(memory_space=pltpu.VMEM),
        ],
        out_specs=pl.BlockSpec(memory_space=pltpu.VMEM),
        scratch_shapes=[
            pltpu.VMEM((LOG_N, H, N_TOK // H, D_OUT), jnp.bfloat16),
            pltpu.VMEM((LOG_N, H, N_TOK // H, D_OUT), jnp.bfloat16),
            pltpu.SemaphoreType.DMA((LOG_N, H)),
            pltpu.SemaphoreType.DMA((LOG_N, H)),
        ],
        compiler_params=pltpu.CompilerParams(collective_id=0),
    )(x, router_W, route_idx, expert_W)


# device time: 17698 ns/iter; 1.0737x vs baseline; 1.0737x over previous
import jax
import jax.numpy as jnp
from jax import lax
from jax.experimental import pallas as pl
from jax.experimental.pallas import tpu as pltpu

N_DEV = 16
LOG_N = 4
E_LOCAL = 2
N_EXPERTS = 32
N_TOK = 256
D_OUT = 256
GROUP_ORDERS = ((0, 1, 2, 3), (2, 3, 0, 1))
G = 2
HC = 2


def kernel(x, router_W, route_idx, expert_W):
    def body(x_ref, rw_ref, idx_ref, ew_ref, out_ref,
             send_ref, recv_ref, send_sems, recv_sems):
        my = lax.axis_index("i")
        partners = [my ^ (1 << b) for b in range(LOG_N)]

        barrier_sem = pltpu.get_barrier_semaphore()
        for p in partners:
            pl.semaphore_signal(barrier_sem, inc=1, device_id=(p,),
                                device_id_type=pl.DeviceIdType.MESH)

        xv = x_ref[:, :]
        scores = jnp.dot(xv, rw_ref[:, :], preferred_element_type=jnp.float32)
        s_max = jnp.max(scores, axis=-1, keepdims=True)
        probs = jnp.exp(scores - s_max)
        probs = probs / jnp.sum(probs, axis=-1, keepdims=True)

        idx0 = idx_ref[:, 0:1]
        idx1 = idx_ref[:, 1:2]
        eids = lax.broadcasted_iota(jnp.int32, (N_TOK, N_EXPERTS), 1)
        g0 = jnp.sum(probs * (eids == idx0).astype(jnp.float32),
                     axis=1, keepdims=True)
        g1 = jnp.sum(probs * (eids == idx1).astype(jnp.float32),
                     axis=1, keepdims=True)
        gs = g0 + g1

        acc = jnp.zeros((N_TOK, D_OUT), jnp.float32)
        for j in range(E_LOCAL):
            e_glob = my * E_LOCAL + j
            p_e = jnp.sum(probs * (eids == e_glob).astype(jnp.float32),
                          axis=1, keepdims=True)
            sel = jnp.logical_or(idx0 == e_glob, idx1 == e_glob)
            gate = jnp.where(sel, p_e / gs, 0.0)
            y = jnp.dot(xv, ew_ref[j, :, :],
                        preferred_element_type=jnp.float32)
            acc = acc + gate * y
        out_ref[:, :] = acc

        pl.semaphore_wait(barrier_sem, LOG_N)

        rows = N_TOK // (G * HC)

        def off(g, h):
            return (g * HC + h) * rows

        def make(g, k, h):
            return pltpu.make_async_remote_copy(
                src_ref=send_ref.at[g, k, h],
                dst_ref=recv_ref.at[g, k, h],
                send_sem=send_sems.at[g, k, h],
                recv_sem=recv_sems.at[g, k, h],
                device_id=(partners[GROUP_ORDERS[g][k]],),
                device_id_type=pl.DeviceIdType.MESH,
            )

        descs = {}
        for h in range(HC):
            for g in range(G):
                send_ref[g, 0, h, :, :] = out_ref[
                    pl.ds(off(g, h), rows), :].astype(jnp.bfloat16)
                descs[(g, 0, h)] = make(g, 0, h)
                descs[(g, 0, h)].start()
        for k in range(LOG_N):
            for h in range(HC):
                for g in range(G):
                    descs[(g, k, h)].wait()
                    out_ref[pl.ds(off(g, h), rows), :] = (
                        out_ref[pl.ds(off(g, h), rows), :]
                        + recv_ref[g, k, h, :, :].astype(jnp.float32)
                    )
                    if k + 1 < LOG_N:
                        send_ref[g, k + 1, h, :, :] = out_ref[
                            pl.ds(off(g, h), rows), :].astype(jnp.bfloat16)
                        descs[(g, k + 1, h)] = make(g, k + 1, h)
                        descs[(g, k + 1, h)].start()

    return pl.pallas_call(
        body,
        out_shape=jax.ShapeDtypeStruct((N_TOK, D_OUT), jnp.float32),
        in_specs=[
            pl.BlockSpec(memory_space=pltpu.VMEM),
            pl.BlockSpec(memory_space=pltpu.VMEM),
            pl.BlockSpec(memory_space=pltpu.VMEM),
            pl.BlockSpec(memory_space=pltpu.VMEM),
        ],
        out_specs=pl.BlockSpec(memory_space=pltpu.VMEM),
        scratch_shapes=[
            pltpu.VMEM((G, LOG_N, HC, N_TOK // (G * HC), D_OUT),
                       jnp.bfloat16),
            pltpu.VMEM((G, LOG_N, HC, N_TOK // (G * HC), D_OUT),
                       jnp.bfloat16),
            pltpu.SemaphoreType.DMA((G, LOG_N, HC)),
            pltpu.SemaphoreType.DMA((G, LOG_N, HC)),
        ],
        compiler_params=pltpu.CompilerParams(collective_id=0),
    )(x, router_W, route_idx, expert_W)


# device time: 17274 ns/iter; 1.1001x vs baseline; 1.0245x over previous
import jax
import jax.numpy as jnp
from jax import lax
from jax.experimental import pallas as pl
from jax.experimental.pallas import tpu as pltpu

N_DEV = 16
LOG_N = 4
E_LOCAL = 2
N_EXPERTS = 32
N_TOK = 256
D_OUT = 256
GROUP_MASKS = ((1, 3, 4, 8), (4, 8, 1, 3))
G = 2
HC = 2


def kernel(x, router_W, route_idx, expert_W):
    def body(x_ref, rw_ref, idx_ref, ew_ref, out_ref,
             send_ref, recv_ref, send_sems, recv_sems):
        my = lax.axis_index("i")
        partners = {m: my ^ m for m in GROUP_MASKS[0]}

        barrier_sem = pltpu.get_barrier_semaphore()
        for p in partners.values():
            pl.semaphore_signal(barrier_sem, inc=1, device_id=(p,),
                                device_id_type=pl.DeviceIdType.MESH)

        xv = x_ref[:, :]
        scores = jnp.dot(xv, rw_ref[:, :], preferred_element_type=jnp.float32)
        s_max = jnp.max(scores, axis=-1, keepdims=True)
        probs = jnp.exp(scores - s_max)
        probs = probs / jnp.sum(probs, axis=-1, keepdims=True)

        idx0 = idx_ref[:, 0:1]
        idx1 = idx_ref[:, 1:2]
        eids = lax.broadcasted_iota(jnp.int32, (N_TOK, N_EXPERTS), 1)
        g0 = jnp.sum(probs * (eids == idx0).astype(jnp.float32),
                     axis=1, keepdims=True)
        g1 = jnp.sum(probs * (eids == idx1).astype(jnp.float32),
                     axis=1, keepdims=True)
        gs = g0 + g1

        gates = []
        for j in range(E_LOCAL):
            e_glob = my * E_LOCAL + j
            p_e = jnp.sum(probs * (eids == e_glob).astype(jnp.float32),
                          axis=1, keepdims=True)
            sel = jnp.logical_or(idx0 == e_glob, idx1 == e_glob)
            gates.append(jnp.where(sel, p_e / gs, 0.0))

        pl.semaphore_wait(barrier_sem, LOG_N)

        rows = N_TOK // (G * HC)

        def off(g, h):
            return (g * HC + h) * rows

        def make(g, k, h):
            return pltpu.make_async_remote_copy(
                src_ref=send_ref.at[g, k, h],
                dst_ref=recv_ref.at[g, k, h],
                send_sem=send_sems.at[g, k, h],
                recv_sem=recv_sems.at[g, k, h],
                device_id=(partners[GROUP_MASKS[g][k]],),
                device_id_type=pl.DeviceIdType.MESH,
            )

        descs = {}
        for h in range(HC):
            for g in range(G):
                o = off(g, h)
                xc = xv[o:o + rows, :]
                acc_c = gates[0][o:o + rows, :] * jnp.dot(
                    xc, ew_ref[0, :, :], preferred_element_type=jnp.float32)
                acc_c = acc_c + gates[1][o:o + rows, :] * jnp.dot(
                    xc, ew_ref[1, :, :], preferred_element_type=jnp.float32)
                out_ref[pl.ds(o, rows), :] = acc_c
                send_ref[g, 0, h, :, :] = acc_c.astype(jnp.bfloat16)
                descs[(g, 0, h)] = make(g, 0, h)
                descs[(g, 0, h)].start()
        for k in range(LOG_N):
            for h in range(HC):
                for g in range(G):
                    descs[(g, k, h)].wait()
                    out_ref[pl.ds(off(g, h), rows), :] = (
                        out_ref[pl.ds(off(g, h), rows), :]
                        + recv_ref[g, k, h, :, :].astype(jnp.float32)
                    )
                    if k + 1 < LOG_N:
                        send_ref[g, k + 1, h, :, :] = out_ref[
                            pl.ds(off(g, h), rows), :].astype(jnp.bfloat16)
                        descs[(g, k + 1, h)] = make(g, k + 1, h)
                        descs[(g, k + 1, h)].start()

    return pl.pallas_call(
        body,
        out_shape=jax.ShapeDtypeStruct((N_TOK, D_OUT), jnp.float32),
        in_specs=[
            pl.BlockSpec(memory_space=pltpu.VMEM),
            pl.BlockSpec(memory_space=pltpu.VMEM),
            pl.BlockSpec(memory_space=pltpu.VMEM),
            pl.BlockSpec(memory_space=pltpu.VMEM),
        ],
        out_specs=pl.BlockSpec(memory_space=pltpu.VMEM),
        scratch_shapes=[
            pltpu.VMEM((G, LOG_N, HC, N_TOK // (G * HC), D_OUT),
                       jnp.bfloat16),
            pltpu.VMEM((G, LOG_N, HC, N_TOK // (G * HC), D_OUT),
                       jnp.bfloat16),
            pltpu.SemaphoreType.DMA((G, LOG_N, HC)),
            pltpu.SemaphoreType.DMA((G, LOG_N, HC)),
        ],
        compiler_params=pltpu.CompilerParams(collective_id=0),
    )(x, router_W, route_idx, expert_W)


# device time: 15592 ns/iter; 1.2188x vs baseline; 1.1079x over previous
import jax
import jax.numpy as jnp
from jax import lax
from jax.experimental import pallas as pl
from jax.experimental.pallas import tpu as pltpu

N_DEV = 16
E_LOCAL = 2
N_EXPERTS = 32
N_TOK = 256
D_OUT = 256

PLANE = (1, 3, 2)
ZLINE = (4, 8, 12)
GROUP_PHASES = ((PLANE, ZLINE), (ZLINE, PLANE))
G = 2
NP = 3
HC = 2


def kernel(x, router_W, route_idx, expert_W):
    def body(x_ref, rw_ref, idx_ref, ew_ref, out_ref,
             send_ref, recv_ref, send_sems, recv_sems):
        my = lax.axis_index("i")
        partner = {m: my ^ m for m in PLANE + ZLINE}

        barrier_sem = pltpu.get_barrier_semaphore()
        for p in partner.values():
            pl.semaphore_signal(barrier_sem, inc=1, device_id=(p,),
                                device_id_type=pl.DeviceIdType.MESH)

        xv = x_ref[:, :]
        scores = jnp.dot(xv, rw_ref[:, :], preferred_element_type=jnp.float32)
        s_max = jnp.max(scores, axis=-1, keepdims=True)
        probs = jnp.exp(scores - s_max)
        probs = probs / jnp.sum(probs, axis=-1, keepdims=True)

        idx0 = idx_ref[:, 0:1]
        idx1 = idx_ref[:, 1:2]
        eids = lax.broadcasted_iota(jnp.int32, (N_TOK, N_EXPERTS), 1)
        g0 = jnp.sum(probs * (eids == idx0).astype(jnp.float32),
                     axis=1, keepdims=True)
        g1 = jnp.sum(probs * (eids == idx1).astype(jnp.float32),
                     axis=1, keepdims=True)
        gs = g0 + g1

        gates = []
        for j in range(E_LOCAL):
            e_glob = my * E_LOCAL + j
            p_e = jnp.sum(probs * (eids == e_glob).astype(jnp.float32),
                          axis=1, keepdims=True)
            sel = jnp.logical_or(idx0 == e_glob, idx1 == e_glob)
            gates.append(jnp.where(sel, p_e / gs, 0.0))

        pl.semaphore_wait(barrier_sem, 2 * NP)

        rows = N_TOK // (G * HC)

        def off(g, h):
            return (g * HC + h) * rows

        def make(g, p, i, h):
            m = GROUP_PHASES[g][p][i]
            return pltpu.make_async_remote_copy(
                src_ref=send_ref.at[g, p, h],
                dst_ref=recv_ref.at[g, p, i, h],
                send_sem=send_sems.at[g, p, i, h],
                recv_sem=recv_sems.at[g, p, i, h],
                device_id=(partner[m],),
                device_id_type=pl.DeviceIdType.MESH,
            )

        descs = {}
        for h in range(HC):
            for g in range(G):
                o = off(g, h)
                xc = xv[o:o + rows, :]
                acc_c = gates[0][o:o + rows, :] * jnp.dot(
                    xc, ew_ref[0, :, :], preferred_element_type=jnp.float32)
                acc_c = acc_c + gates[1][o:o + rows, :] * jnp.dot(
                    xc, ew_ref[1, :, :], preferred_element_type=jnp.float32)
                out_ref[pl.ds(o, rows), :] = acc_c
                send_ref[g, 0, h, :, :] = acc_c.astype(jnp.bfloat16)
                for i in range(NP):
                    descs[(g, 0, i, h)] = make(g, 0, i, h)
                    descs[(g, 0, i, h)].start()

        for p in range(2):
            for h in range(HC):
                for g in range(G):
                    o = off(g, h)
                    s = out_ref[pl.ds(o, rows), :]
                    for i in range(NP):
                        descs[(g, p, i, h)].wait()
                        s = s + recv_ref[g, p, i, h, :, :].astype(jnp.float32)
                    out_ref[pl.ds(o, rows), :] = s
                    if p == 0:
                        send_ref[g, 1, h, :, :] = s.astype(jnp.bfloat16)
                        for i in range(NP):
                            descs[(g, 1, i, h)] = make(g, 1, i, h)
                            descs[(g, 1, i, h)].start()

    return pl.pallas_call(
        body,
        out_shape=jax.ShapeDtypeStruct((N_TOK, D_OUT), jnp.float32),
        in_specs=[
            pl.BlockSpec(memory_space=pltpu.VMEM),
            pl.BlockSpec(memory_space=pltpu.VMEM),
            pl.BlockSpec(memory_space=pltpu.VMEM),
            pl.BlockSpec(memory_space=pltpu.VMEM),
        ],
        out_specs=pl.BlockSpec(memory_space=pltpu.VMEM),
        scratch_shapes=[
            pltpu.VMEM((G, 2, HC, N_TOK // (G * HC), D_OUT),
                       jnp.bfloat16),
            pltpu.VMEM((G, 2, NP, HC, N_TOK // (G * HC), D_OUT),
                       jnp.bfloat16),
            pltpu.SemaphoreType.DMA((G, 2, NP, HC)),
            pltpu.SemaphoreType.DMA((G, 2, NP, HC)),
        ],
        compiler_params=pltpu.CompilerParams(collective_id=0),
    )(x, router_W, route_idx, expert_W)
